# baseline (speedup 1.0000x reference)
import jax
import jax.numpy as jnp
from jax.experimental import pallas as pl
from jax.experimental.pallas import tpu as pltpu

N_TAB_ = 10000
NR_ = 5
EMB_ = 32
B_ = 1024
SPLIT_ = 5
SUBCHK_ = 200
CHK_ = SPLIT_ * SUBCHK_
NK_ = N_TAB_ // CHK_
GCHUNK_ = 500
NGC_ = N_TAB_ // GCHUNK_


def _dot_t(a, w):
    return jax.lax.dot_general(a, w, (((1,), (1,)), ((), ())),
                               preferred_element_type=jnp.float32)


def _dot_tl(et, tab):
    return jax.lax.dot_general(et, tab, (((0,), (0,)), ((), ())),
                               preferred_element_type=jnp.float32)


def _gcmc_body(*refs):
    (x_ref, *edge_refs, utab_ref, itab_ref,
     guW_ref, gub_ref, giW_ref, gib_ref,
     f1uW_ref, f1ub_ref, f1iW_ref, f1ib_ref,
     l1W_ref, l1b_ref, l2W_ref, l2b_ref, l3W_ref, l3b_ref,
     out_ref, au_scr, ai_scr, ug_scr, ig_scr) = refs
    eUIT_refs = edge_refs[:SPLIT_]
    eIUT_refs = edge_refs[SPLIT_:]
    n = pl.program_id(0)
    k = pl.program_id(1)
    t = n * NK_ + k

    hu = hi = None
    for s in range(SPLIT_):
        off = pl.ds(k * CHK_ + s * SUBCHK_, SUBCHK_)
        pu = _dot_tl(eIUT_refs[s][0], utab_ref[off, :])
        pi = _dot_tl(eUIT_refs[s][0], itab_ref[off, :])
        hu = pu if hu is None else hu + pu
        hi = pi if hi is None else hi + pi

    @pl.when(k == 0)
    def _():
        au_scr[n] = hu
        ai_scr[n] = hi

    @pl.when(k != 0)
    def _():
        au_scr[n] += hu
        ai_scr[n] += hi

    uid = x_ref[:, 0:1]
    iid = x_ref[:, 1:2]
    for c in range(2 * NGC_):
        tab_ref, idx, g_scr = ((utab_ref, uid, ug_scr) if c < NGC_
                               else (itab_ref, iid, ig_scr))
        chunk = c % NGC_

        @pl.when(t == c + 1)
        def _(tab_ref=tab_ref, idx=idx, g_scr=g_scr, chunk=chunk):
            base = chunk * GCHUNK_
            ids = jax.lax.broadcasted_iota(
                jnp.int32, (B_, GCHUNK_), 1) + base
            m = (ids == idx).astype(jnp.float32)
            g = jnp.dot(m, tab_ref[pl.ds(base, GCHUNK_), :],
                        preferred_element_type=jnp.float32)
            if chunk == 0:
                g_scr[...] = g
            else:
                g_scr[...] += g

    @pl.when(t == NR_ * NK_ - 1)
    def _epilogue():
        gu_h = jnp.concatenate(
            [jnp.maximum(_dot_t(au_scr[m][:, :EMB_], guW_ref[...])
                         + gub_ref[...], 0.0) for m in range(NR_)], axis=1)
        gi_h = jnp.concatenate(
            [jnp.maximum(_dot_t(ai_scr[m][:, :EMB_], giW_ref[...])
                         + gib_ref[...], 0.0) for m in range(NR_)], axis=1)
        guo = _dot_t(gu_h, f1uW_ref[...]) + f1ub_ref[...]
        gio = _dot_t(gi_h, f1iW_ref[...]) + f1ib_ref[...]

        ue_g = ug_scr[:, :EMB_]
        ub_g = ug_scr[:, EMB_:EMB_ + 1]
        ie_g = ig_scr[:, :EMB_]
        ib_g = ig_scr[:, EMB_:EMB_ + 1]

        h = jnp.concatenate(
            [ue_g * ie_g, ue_g * gio, guo * ie_g, guo * gio], axis=1)
        x1 = jnp.maximum(_dot_t(h, l1W_ref[...]) + l1b_ref[...], 0.0)
        x2 = jnp.maximum(_dot_t(x1, l2W_ref[...]) + l2b_ref[...], 0.0)
        x3 = jnp.sum(x2 * l3W_ref[...], axis=1, keepdims=True)
        x3 = x3 + l3b_ref[0, 0]
        out_ref[...] = x3 + ub_g + ib_g


def kernel(x, edge_UI, edge_IU, user_embedding, item_embedding,
           GCN_user_W, GCN_user_b, GCN_item_W, GCN_item_b,
           fc1_user_W, fc1_user_b, fc1_item_W, fc1_item_b,
           l1_W, l1_b, l2_W, l2_b, l3_W, l3_b,
           user_bias, item_bias):
    full = lambda a: pl.BlockSpec(a.shape, lambda n, k: (0,) * a.ndim)
    row2 = lambda v: v.reshape(1, -1)

    eUIT = jnp.swapaxes(edge_UI, 1, 2)
    eIUT = jnp.swapaxes(edge_IU, 1, 2)

    utab = jnp.concatenate([user_embedding, user_bias], axis=1)
    itab = jnp.concatenate([item_embedding, item_bias], axis=1)

    def edge_spec(s):
        return pl.BlockSpec(
            (1, SUBCHK_, B_),
            lambda n, k, s=s: (n, k * SPLIT_ + s, 0))

    out = pl.pallas_call(
        _gcmc_body,
        grid=(NR_, NK_),
        in_specs=[
            pl.BlockSpec((B_, 2), lambda n, k: (0, 0)),
            *[edge_spec(s) for s in range(SPLIT_)],
            *[edge_spec(s) for s in range(SPLIT_)],
            full(utab), full(itab),
            full(GCN_user_W), full(row2(GCN_user_b)),
            full(GCN_item_W), full(row2(GCN_item_b)),
            full(fc1_user_W), full(row2(fc1_user_b)),
            full(fc1_item_W), full(row2(fc1_item_b)),
            full(l1_W), full(row2(l1_b)),
            full(l2_W), full(row2(l2_b)),
            full(l3_W), full(row2(l3_b)),
        ],
        out_specs=pl.BlockSpec((B_, 1), lambda n, k: (0, 0)),
        out_shape=jax.ShapeDtypeStruct((B_, 1), jnp.float32),
        scratch_shapes=[
            pltpu.VMEM((NR_, B_, EMB_ + 1), jnp.float32),
            pltpu.VMEM((NR_, B_, EMB_ + 1), jnp.float32),
            pltpu.VMEM((B_, EMB_ + 1), jnp.float32),
            pltpu.VMEM((B_, EMB_ + 1), jnp.float32),
        ],
        compiler_params=pltpu.CompilerParams(
            dimension_semantics=("arbitrary", "arbitrary")),
    )(x, *([eUIT] * SPLIT_), *([eIUT] * SPLIT_), utab, itab,
      GCN_user_W, row2(GCN_user_b), GCN_item_W, row2(GCN_item_b),
      fc1_user_W, row2(fc1_user_b), fc1_item_W, row2(fc1_item_b),
      l1_W, row2(l1_b), l2_W, row2(l2_b), l3_W, row2(l3_b))
    return out.reshape(-1)


# speedup vs baseline: 1.2447x; 1.2447x over previous
import jax
import jax.numpy as jnp
from jax.experimental import pallas as pl
from jax.experimental.pallas import tpu as pltpu

N_TAB_ = 10000
NR_ = 5
EMB_ = 32
B_ = 1024
SPLIT_ = 2
SUBCHK_ = 1000
CHK_ = SPLIT_ * SUBCHK_
NK_ = N_TAB_ // CHK_
GCHUNK_ = 1000
NGC_ = N_TAB_ // GCHUNK_


def _dot_t(a, w):
    return jax.lax.dot_general(a, w, (((1,), (1,)), ((), ())),
                               preferred_element_type=jnp.float32)


def _dot_tl(et, tab):
    return jax.lax.dot_general(et, tab, (((0,), (0,)), ((), ())),
                               preferred_element_type=jnp.float32)


def _gcmc_body(*refs):
    (x_ref, *edge_refs, utab_ref, itab_ref,
     guW_ref, gub_ref, giW_ref, gib_ref,
     f1uW_ref, f1ub_ref, f1iW_ref, f1ib_ref,
     l1W_ref, l1b_ref, l2W_ref, l2b_ref, l3W_ref, l3b_ref,
     out_ref, au_scr, ai_scr, ug_scr, ig_scr) = refs
    eUIT_refs = edge_refs[:SPLIT_]
    eIUT_refs = edge_refs[SPLIT_:]
    n = pl.program_id(0)
    k = pl.program_id(1)
    t = n * NK_ + k

    hu = hi = None
    for s in range(SPLIT_):
        off = pl.ds(k * CHK_ + s * SUBCHK_, SUBCHK_)
        pu = _dot_tl(eIUT_refs[s][0], utab_ref[off, :])
        pi = _dot_tl(eUIT_refs[s][0], itab_ref[off, :])
        hu = pu if hu is None else hu + pu
        hi = pi if hi is None else hi + pi

    @pl.when(k == 0)
    def _():
        au_scr[n] = hu
        ai_scr[n] = hi

    @pl.when(k != 0)
    def _():
        au_scr[n] += hu
        ai_scr[n] += hi

    uid = x_ref[:, 0:1]
    iid = x_ref[:, 1:2]
    for c in range(2 * NGC_):
        tab_ref, idx, g_scr = ((utab_ref, uid, ug_scr) if c < NGC_
                               else (itab_ref, iid, ig_scr))
        chunk = c % NGC_

        @pl.when(t == c + 1)
        def _(tab_ref=tab_ref, idx=idx, g_scr=g_scr, chunk=chunk):
            base = chunk * GCHUNK_
            ids = jax.lax.broadcasted_iota(
                jnp.int32, (B_, GCHUNK_), 1) + base
            m = (ids == idx).astype(jnp.float32)
            g = jnp.dot(m, tab_ref[pl.ds(base, GCHUNK_), :],
                        preferred_element_type=jnp.float32)
            if chunk == 0:
                g_scr[...] = g
            else:
                g_scr[...] += g

    @pl.when(t == NR_ * NK_ - 1)
    def _epilogue():
        gu_h = jnp.concatenate(
            [jnp.maximum(_dot_t(au_scr[m][:, :EMB_], guW_ref[...])
                         + gub_ref[...], 0.0) for m in range(NR_)], axis=1)
        gi_h = jnp.concatenate(
            [jnp.maximum(_dot_t(ai_scr[m][:, :EMB_], giW_ref[...])
                         + gib_ref[...], 0.0) for m in range(NR_)], axis=1)
        guo = _dot_t(gu_h, f1uW_ref[...]) + f1ub_ref[...]
        gio = _dot_t(gi_h, f1iW_ref[...]) + f1ib_ref[...]

        ue_g = ug_scr[:, :EMB_]
        ub_g = ug_scr[:, EMB_:EMB_ + 1]
        ie_g = ig_scr[:, :EMB_]
        ib_g = ig_scr[:, EMB_:EMB_ + 1]

        h = jnp.concatenate(
            [ue_g * ie_g, ue_g * gio, guo * ie_g, guo * gio], axis=1)
        x1 = jnp.maximum(_dot_t(h, l1W_ref[...]) + l1b_ref[...], 0.0)
        x2 = jnp.maximum(_dot_t(x1, l2W_ref[...]) + l2b_ref[...], 0.0)
        x3 = jnp.sum(x2 * l3W_ref[...], axis=1, keepdims=True)
        x3 = x3 + l3b_ref[0, 0]
        out_ref[...] = x3 + ub_g + ib_g


def kernel(x, edge_UI, edge_IU, user_embedding, item_embedding,
           GCN_user_W, GCN_user_b, GCN_item_W, GCN_item_b,
           fc1_user_W, fc1_user_b, fc1_item_W, fc1_item_b,
           l1_W, l1_b, l2_W, l2_b, l3_W, l3_b,
           user_bias, item_bias):
    full = lambda a: pl.BlockSpec(a.shape, lambda n, k: (0,) * a.ndim)
    row2 = lambda v: v.reshape(1, -1)

    eUIT = jnp.swapaxes(edge_UI, 1, 2)
    eIUT = jnp.swapaxes(edge_IU, 1, 2)

    utab = jnp.concatenate([user_embedding, user_bias], axis=1)
    itab = jnp.concatenate([item_embedding, item_bias], axis=1)

    def edge_spec(s):
        return pl.BlockSpec(
            (1, SUBCHK_, B_),
            lambda n, k, s=s: (n, k * SPLIT_ + s, 0))

    out = pl.pallas_call(
        _gcmc_body,
        grid=(NR_, NK_),
        in_specs=[
            pl.BlockSpec((B_, 2), lambda n, k: (0, 0)),
            *[edge_spec(s) for s in range(SPLIT_)],
            *[edge_spec(s) for s in range(SPLIT_)],
            full(utab), full(itab),
            full(GCN_user_W), full(row2(GCN_user_b)),
            full(GCN_item_W), full(row2(GCN_item_b)),
            full(fc1_user_W), full(row2(fc1_user_b)),
            full(fc1_item_W), full(row2(fc1_item_b)),
            full(l1_W), full(row2(l1_b)),
            full(l2_W), full(row2(l2_b)),
            full(l3_W), full(row2(l3_b)),
        ],
        out_specs=pl.BlockSpec((B_, 1), lambda n, k: (0, 0)),
        out_shape=jax.ShapeDtypeStruct((B_, 1), jnp.float32),
        scratch_shapes=[
            pltpu.VMEM((NR_, B_, EMB_ + 1), jnp.float32),
            pltpu.VMEM((NR_, B_, EMB_ + 1), jnp.float32),
            pltpu.VMEM((B_, EMB_ + 1), jnp.float32),
            pltpu.VMEM((B_, EMB_ + 1), jnp.float32),
        ],
        compiler_params=pltpu.CompilerParams(
            dimension_semantics=("arbitrary", "arbitrary")),
    )(x, *([eUIT] * SPLIT_), *([eIUT] * SPLIT_), utab, itab,
      GCN_user_W, row2(GCN_user_b), GCN_item_W, row2(GCN_item_b),
      fc1_user_W, row2(fc1_user_b), fc1_item_W, row2(fc1_item_b),
      l1_W, row2(l1_b), l2_W, row2(l2_b), l3_W, row2(l3_b))
    return out.reshape(-1)
